# baseline (speedup 1.0000x reference)
import functools

import jax
import jax.numpy as jnp
from jax import lax
from jax.experimental import pallas as pl
from jax.experimental.pallas import tpu as pltpu
from jax.experimental.pallas import tpu_sc as plsc

N = 10000
E = 320000
D = 128

NC = 2
NS = 16
NW = NC * NS
EPW = E // NW
W1 = 80
EH = E // 2
EPT1 = EH // NS
NWIN1 = EPT1 // W1
W3 = 80
NWIN3 = EPW // W3
NPAD = 10240
HW = D // 2

_f32 = jnp.float32


def _k0_body(nf_ref, ws_ref, wr_ref, be1_ref, out_ref):
    x = nf_ref[...]
    out_ref[0] = jnp.dot(x, ws_ref[...], preferred_element_type=_f32)
    out_ref[1] = (jnp.dot(x, wr_ref[...], preferred_element_type=_f32)
                  + be1_ref[...])


def _node_proj(nf, ws, wr, be1):
    bn = 400
    grid = (N // bn,)
    return pl.pallas_call(
        _k0_body,
        grid=grid,
        in_specs=[
            pl.BlockSpec((bn, D), lambda i: (i, 0)),
            pl.BlockSpec((D, D), lambda i: (0, 0)),
            pl.BlockSpec((D, D), lambda i: (0, 0)),
            pl.BlockSpec((1, D), lambda i: (0, 0)),
        ],
        out_specs=pl.BlockSpec((NC, bn, D), lambda i: (0, i, 0)),
        out_shape=jax.ShapeDtypeStruct((NC, NPAD, D), _f32),
    )(nf, ws, wr, be1)


def _k1_body_shell(EBASE, np2_hbm, sr_hbm, g2_hbm,
                   idx0, rows0, idx1, rows1, isem0, isem1, wsem0, wsem1,
                   table):
    cid = lax.axis_index("c")
    sid = lax.axis_index("s")
    rpt = NPAD // NS

    pltpu.sync_copy(np2_hbm.at[cid, pl.ds(sid * rpt, rpt)],
                    table.at[pl.ds(sid * rpt, rpt)])
    plsc.subcore_barrier()

    base = EBASE + sid * EPT1
    bufs = ((idx0, rows0), (idx1, rows1))
    isems = (isem0, isem1)
    wsems = (wsem0, wsem1)

    def fire_load_idx(j, b):
        pltpu.async_copy(sr_hbm.at[pl.ds(cid * E + base + j * W1, W1)],
                         bufs[b][0], isems[b])

    def gather(b):
        idx, rows = bufs[b]
        pltpu.make_async_copy(sr_hbm.at[pl.ds(cid * E + base, W1)],
                              idx, isems[b]).wait()
        pltpu.sync_copy(table.at[idx], rows)

    def fire_wb(j, b):
        pltpu.async_copy(bufs[b][1],
                         g2_hbm.at[cid, pl.ds(base + j * W1, W1)], wsems[b])

    def wait_wb(j, b):
        pltpu.make_async_copy(bufs[b][1],
                              g2_hbm.at[cid, pl.ds(base + j * W1, W1)],
                              wsems[b]).wait()

    fire_load_idx(0, 0)

    def body(i, carry):
        j = i * 2
        fire_load_idx(j + 1, 1)
        gather(0)
        fire_wb(j, 0)
        @pl.when(j + 2 < NWIN1)
        def _():
            fire_load_idx(j + 2, 0)
        gather(1)
        wait_wb(j, 0)
        fire_wb(j + 1, 1)
        wait_wb(j + 1, 1)
        return carry

    lax.fori_loop(0, NWIN1 // 2, body, 0)
    if NWIN1 % 2 == 1:
        gather(0)
        fire_wb(NWIN1 - 1, 0)
        wait_wb(NWIN1 - 1, 0)


def _gather(np2, sr, ebase):
    body = functools.partial(_k1_body_shell, ebase)
    mesh = plsc.VectorSubcoreMesh(core_axis_name="c", subcore_axis_name="s",
                                  num_cores=NC, num_subcores=NS)
    return pl.kernel(
        body,
        out_type=jax.ShapeDtypeStruct((NC, E, D), _f32),
        mesh=mesh,
        scratch_types=[
            pltpu.VMEM((W1,), jnp.int32),
            pltpu.VMEM((W1, D), _f32),
            pltpu.VMEM((W1,), jnp.int32),
            pltpu.VMEM((W1, D), _f32),
            pltpu.SemaphoreType.DMA,
            pltpu.SemaphoreType.DMA,
            pltpu.SemaphoreType.DMA,
            pltpu.SemaphoreType.DMA,
            pltpu.VMEM_SHARED((NPAD, D), _f32),
        ],
    )(np2, sr)


def _k2_body(gs_ref, gr_ref, ef_ref, pd_ref, wd_ref, we_ref, we2_ref,
             be2_ref, wa_ref, wat_ref, ba_ref, ne_ref, p_ref, w_ref):
    bf16 = jnp.bfloat16
    ef = ef_ref[...]
    h = gs_ref[0] + gr_ref[0] + jnp.dot(
        ef.astype(bf16), we_ref[...].astype(bf16), preferred_element_type=_f32)
    h = jnp.maximum(h, 0.0)
    ne = jnp.dot(h.astype(bf16), we2_ref[...].astype(bf16),
                 preferred_element_type=_f32) + be2_ref[...]
    wab = jnp.broadcast_to(wa_ref[...], (D, D)).astype(bf16)
    a_full = jnp.dot(ne.astype(bf16), wab,
                     preferred_element_type=_f32) + ba_ref[...]
    a_full = jnp.where(a_full >= 0, a_full, 0.2 * a_full)
    wf = jnp.exp(a_full)
    p_ref[...] = ne * wf
    ne_ref[...] = ne + ef
    wft = wf.T
    w_ref[...] = wft[0:1, :].reshape(1, 1, -1)


def _edge_mlp_half(g2, efne, p_donor, w_donor, we, we2, be2, wa, wat, ba, hb):
    be = 2000
    hbb = EH // be
    off = hb * hbb
    grid = (hbb,)
    return pl.pallas_call(
        _k2_body,
        grid=grid,
        in_specs=[
            pl.BlockSpec((1, be, D), lambda i: (0, i + off, 0)),
            pl.BlockSpec((1, be, D), lambda i: (1, i + off, 0)),
            pl.BlockSpec((be, D), lambda i: (i + off, 0)),
            pl.BlockSpec(memory_space=pl.ANY),
            pl.BlockSpec(memory_space=pl.ANY),
            pl.BlockSpec((D, D), lambda i: (0, 0)),
            pl.BlockSpec((D, D), lambda i: (0, 0)),
            pl.BlockSpec((1, D), lambda i: (0, 0)),
            pl.BlockSpec((D, 1), lambda i: (0, 0)),
            pl.BlockSpec((1, D), lambda i: (0, 0)),
            pl.BlockSpec((1, 1), lambda i: (0, 0)),
        ],
        out_specs=[
            pl.BlockSpec((be, D), lambda i: (i + off, 0)),
            pl.BlockSpec((be, D), lambda i: (i + off, 0)),
            pl.BlockSpec((1, 1, be), lambda i: (i + off, 0, 0)),
        ],
        out_shape=[
            jax.ShapeDtypeStruct((E, D), _f32),
            jax.ShapeDtypeStruct((E, D), _f32),
            jax.ShapeDtypeStruct((E // be, 1, be), _f32),
        ],
        input_output_aliases={2: 0, 3: 1, 4: 2},
    )(g2, g2, efne, p_donor, w_donor, we, we2, be2, wa, wat, ba)


def _k3_body(p_hbm, w_hbm, rcv_hbm, zs_hbm, zd_hbm, s_out, d_out,
             ridx0, prows0, wchunk0, ridx1, prows1, wchunk1,
             lsem0, lsem1, ssem0, ssem1, s_acc, d_acc):
    cid = lax.axis_index("c")
    sid = lax.axis_index("s")
    wid = sid * NC + cid
    rows_per_tile = NPAD // NS

    pltpu.sync_copy(zs_hbm.at[pl.ds(sid * rows_per_tile, rows_per_tile)],
                    s_acc.at[pl.ds(sid * rows_per_tile, rows_per_tile)])
    pltpu.sync_copy(zd_hbm.at[pl.ds(sid * rows_per_tile, rows_per_tile)],
                    d_acc.at[pl.ds(sid * rows_per_tile, rows_per_tile)])
    plsc.subcore_barrier()

    base = wid * EPW
    bufs = ((ridx0, prows0, wchunk0), (ridx1, prows1, wchunk1))
    lsems = (lsem0, lsem1)
    ssems = (ssem0, ssem1)

    def fire_load(j, b):
        ridx, prows, wchunk = bufs[b]
        off = base + j * W3
        pltpu.async_copy(rcv_hbm.at[pl.ds(off, W3)], ridx, lsems[b])
        pltpu.async_copy(p_hbm.at[pl.ds(off, W3)], prows, lsems[b])
        pltpu.async_copy(w_hbm.at[pl.ds(off, W3)], wchunk, lsems[b])

    def wait_load(j, b):
        ridx, prows, wchunk = bufs[b]
        off = base + j * W3
        pltpu.make_async_copy(rcv_hbm.at[pl.ds(off, W3)], ridx, lsems[b]).wait()
        pltpu.make_async_copy(p_hbm.at[pl.ds(off, W3)], prows, lsems[b]).wait()
        pltpu.make_async_copy(w_hbm.at[pl.ds(off, W3)], wchunk, lsems[b]).wait()

    def fire_scatter(b):
        ridx, prows, wchunk = bufs[b]
        pltpu.async_copy(prows, s_acc.at[ridx], ssems[b], add=True)
        pltpu.async_copy(wchunk, d_acc.at[ridx], ssems[b], add=True)

    def wait_scatter(b):
        ridx, prows, wchunk = bufs[b]
        pltpu.make_async_copy(prows, s_acc.at[ridx], ssems[b]).wait()
        pltpu.make_async_copy(wchunk, d_acc.at[ridx], ssems[b]).wait()

    fire_load(0, 0)

    def body(i, carry):
        j = i * 2
        @pl.when(j >= 1)
        def _():
            wait_scatter(1)
        @pl.when(j + 1 < NWIN3)
        def _():
            fire_load(j + 1, 1)
        wait_load(j, 0)
        fire_scatter(0)
        @pl.when(j + 1 < NWIN3)
        def _():
            wait_scatter(0)
            @pl.when(j + 2 < NWIN3)
            def _():
                fire_load(j + 2, 0)
            wait_load(j + 1, 1)
            fire_scatter(1)
        return carry

    lax.fori_loop(0, (NWIN3 + 1) // 2, body, 0)
    wait_scatter(0)
    if NWIN3 % 2 == 0:
        wait_scatter(1)
    plsc.subcore_barrier()

    pltpu.sync_copy(s_acc.at[pl.ds(sid * rows_per_tile, rows_per_tile)],
                    s_out.at[cid, pl.ds(sid * rows_per_tile, rows_per_tile)])
    pltpu.sync_copy(d_acc.at[pl.ds(sid * rows_per_tile, rows_per_tile)],
                    d_out.at[cid, pl.ds(sid * rows_per_tile, rows_per_tile)])


def _segment_sums(p, w, rcv):
    mesh = plsc.VectorSubcoreMesh(core_axis_name="c", subcore_axis_name="s",
                                  num_cores=NC, num_subcores=NS)
    zs = jnp.zeros((NPAD, D), _f32)
    zd = jnp.zeros((NPAD,), _f32)
    return pl.kernel(
        _k3_body,
        out_type=(
            jax.ShapeDtypeStruct((NC, NPAD, D), _f32),
            jax.ShapeDtypeStruct((NC, NPAD), _f32),
        ),
        mesh=mesh,
        scratch_types=[
            pltpu.VMEM((W3,), jnp.int32),
            pltpu.VMEM((W3, D), _f32),
            pltpu.VMEM((W3,), _f32),
            pltpu.VMEM((W3,), jnp.int32),
            pltpu.VMEM((W3, D), _f32),
            pltpu.VMEM((W3,), _f32),
            pltpu.SemaphoreType.DMA,
            pltpu.SemaphoreType.DMA,
            pltpu.SemaphoreType.DMA,
            pltpu.SemaphoreType.DMA,
            pltpu.VMEM_SHARED((NPAD, D), _f32),
            pltpu.VMEM_SHARED((NPAD,), _f32),
        ],
    )(p, w, rcv, zs, zd)


def _k4_body(nf_ref, s_ref, d_ref, wn1a_ref, wn1b_ref,
             bn1_ref, wn2_ref, bn2_ref, out_ref):
    x = nf_ref[...]
    d = d_ref[0] + d_ref[1] + 1e-16
    agg = (s_ref[0] + s_ref[1]) / d
    nh = (jnp.dot(x.astype(jnp.bfloat16), wn1a_ref[...].astype(jnp.bfloat16),
                  preferred_element_type=_f32)
          + jnp.dot(agg.astype(jnp.bfloat16),
                    wn1b_ref[...].astype(jnp.bfloat16),
                    preferred_element_type=_f32)
          + bn1_ref[...])
    nh = jnp.maximum(nh, 0.0)
    out_ref[...] = jnp.dot(nh.astype(jnp.bfloat16),
                           wn2_ref[...].astype(jnp.bfloat16),
                           preferred_element_type=_f32) + bn2_ref[...] + x


def _node_mlp(nf, s_part, d_part3, wn1a, wn1b, bn1, wn2, bn2):
    bn = 400
    grid = (N // bn,)
    return pl.pallas_call(
        _k4_body,
        grid=grid,
        in_specs=[
            pl.BlockSpec((bn, D), lambda i: (i, 0)),
            pl.BlockSpec((NC, bn, D), lambda i: (0, i, 0)),
            pl.BlockSpec((NC, bn, 1), lambda i: (0, i, 0)),
            pl.BlockSpec((D, D), lambda i: (0, 0)),
            pl.BlockSpec((D, D), lambda i: (0, 0)),
            pl.BlockSpec((1, D), lambda i: (0, 0)),
            pl.BlockSpec((D, D), lambda i: (0, 0)),
            pl.BlockSpec((1, D), lambda i: (0, 0)),
        ],
        out_specs=pl.BlockSpec((bn, D), lambda i: (i, 0)),
        out_shape=jax.ShapeDtypeStruct((N, D), _f32),
    )(nf, s_part, d_part3, wn1a, wn1b, bn1, wn2, bn2)


def kernel(node_features, edge_features, We1, be1, We2, be2,
           Wn1, bn1, Wn2, bn2, Wa, ba, senders, receivers):
    nf = node_features.reshape(N, D)
    ef = edge_features.reshape(E, D)
    snd = senders.astype(jnp.int32)
    rcv = receivers.astype(jnp.int32)

    ws = We1[:D]
    wr = We1[D:2 * D]
    we = We1[2 * D:]
    be1r = be1.reshape(1, D)
    be2r = be2.reshape(1, D)
    wat = Wa.reshape(1, D)
    bar = ba.reshape(1, 1)
    wn1a = Wn1[:D]
    wn1b = Wn1[D:]
    bn1r = bn1.reshape(1, D)
    bn2r = bn2.reshape(1, D)

    np2 = _node_proj(nf, ws, wr, be1r)
    sr = jnp.concatenate([snd, rcv], axis=0)
    g2a = _gather(np2, sr, 0)
    g2b = _gather(np2, sr, EH)
    pz = jnp.zeros((E, D), _f32)
    wz = jnp.zeros((E // 2000, 1, 2000), _f32)
    ne_a, p_a, w_a = _edge_mlp_half(g2a, ef, pz, wz,
                                    we, We2, be2r, Wa, wat, bar, 0)
    ne, p, w = _edge_mlp_half(g2b, ne_a, p_a, w_a,
                              we, We2, be2r, Wa, wat, bar, 1)
    s_part, d_part = _segment_sums(p, w.reshape(E), rcv)
    new_node = _node_mlp(nf, s_part, d_part.reshape(NC, NPAD, 1),
                         wn1a, wn1b, bn1r, Wn2, bn2r)

    return (new_node.reshape(1, N, D), ne.reshape(1, E, D))


# speedup vs baseline: 1.1783x; 1.1783x over previous
import functools

import jax
import jax.numpy as jnp
from jax import lax
from jax.experimental import pallas as pl
from jax.experimental.pallas import tpu as pltpu
from jax.experimental.pallas import tpu_sc as plsc

N = 10000
E = 320000
D = 128

NC = 2
NS = 16
NW = NC * NS
EPW = E // NW
W1 = 80
EPT1 = E // NS
NWIN1 = EPT1 // W1
W3 = 80
NWIN3 = EPW // W3
NPAD = 10240
HW = D // 2

_f32 = jnp.float32


def _k0_body(nf_ref, ws_ref, wr_ref, be1_ref, out_ref):
    x = nf_ref[...]
    out_ref[0] = jnp.dot(x, ws_ref[...], preferred_element_type=_f32)
    out_ref[1] = (jnp.dot(x, wr_ref[...], preferred_element_type=_f32)
                  + be1_ref[...])


def _node_proj(nf, ws, wr, be1):
    bn = 400
    grid = (N // bn,)
    return pl.pallas_call(
        _k0_body,
        grid=grid,
        in_specs=[
            pl.BlockSpec((bn, D), lambda i: (i, 0)),
            pl.BlockSpec((D, D), lambda i: (0, 0)),
            pl.BlockSpec((D, D), lambda i: (0, 0)),
            pl.BlockSpec((1, D), lambda i: (0, 0)),
        ],
        out_specs=pl.BlockSpec((NC, bn, D), lambda i: (0, i, 0)),
        out_shape=jax.ShapeDtypeStruct((NC, NPAD, D), _f32),
    )(nf, ws, wr, be1)


def _k1_body(np2_hbm, sr_hbm, g2_hbm,
             idx0, rows0, idx1, rows1, isem0, isem1, wsem0, wsem1, table):
    cid = lax.axis_index("c")
    sid = lax.axis_index("s")
    rpt = NPAD // NS

    pltpu.sync_copy(np2_hbm.at[cid, pl.ds(sid * rpt, rpt)],
                    table.at[pl.ds(sid * rpt, rpt)])
    plsc.subcore_barrier()

    base = sid * EPT1
    bufs = ((idx0, rows0), (idx1, rows1))
    isems = (isem0, isem1)
    wsems = (wsem0, wsem1)

    def fire_load_idx(j, b):
        pltpu.async_copy(sr_hbm.at[pl.ds(cid * E + base + j * W1, W1)],
                         bufs[b][0], isems[b])

    def gather(b):
        idx, rows = bufs[b]
        pltpu.make_async_copy(sr_hbm.at[pl.ds(cid * E + base, W1)],
                              idx, isems[b]).wait()
        pltpu.sync_copy(table.at[idx], rows)

    def fire_wb(j, b):
        pltpu.async_copy(bufs[b][1],
                         g2_hbm.at[cid, pl.ds(base + j * W1, W1)], wsems[b])

    def wait_wb(j, b):
        pltpu.make_async_copy(bufs[b][1],
                              g2_hbm.at[cid, pl.ds(base + j * W1, W1)],
                              wsems[b]).wait()

    fire_load_idx(0, 0)

    def body(i, carry):
        j = i * 2
        fire_load_idx(j + 1, 1)
        gather(0)
        fire_wb(j, 0)
        @pl.when(j + 2 < NWIN1)
        def _():
            fire_load_idx(j + 2, 0)
        gather(1)
        wait_wb(j, 0)
        fire_wb(j + 1, 1)
        wait_wb(j + 1, 1)
        return carry

    lax.fori_loop(0, NWIN1 // 2, body, 0)


def _gather(np2, sr):
    mesh = plsc.VectorSubcoreMesh(core_axis_name="c", subcore_axis_name="s",
                                  num_cores=NC, num_subcores=NS)
    return pl.kernel(
        _k1_body,
        out_type=jax.ShapeDtypeStruct((NC, E, D), _f32),
        mesh=mesh,
        scratch_types=[
            pltpu.VMEM((W1,), jnp.int32),
            pltpu.VMEM((W1, D), _f32),
            pltpu.VMEM((W1,), jnp.int32),
            pltpu.VMEM((W1, D), _f32),
            pltpu.SemaphoreType.DMA,
            pltpu.SemaphoreType.DMA,
            pltpu.SemaphoreType.DMA,
            pltpu.SemaphoreType.DMA,
            pltpu.VMEM_SHARED((NPAD, D), _f32),
        ],
    )(np2, sr)


def _k2_body(gs_ref, gr_ref, ef_ref, we_ref, we2_ref, be2_ref, wa_ref,
             wat_ref, ba_ref, ne_ref, p_ref, w_ref):
    bf16 = jnp.bfloat16
    ef = ef_ref[...]
    h = gs_ref[0] + gr_ref[0] + jnp.dot(
        ef.astype(bf16), we_ref[...].astype(bf16), preferred_element_type=_f32)
    h = jnp.maximum(h, 0.0)
    ne = jnp.dot(h.astype(bf16), we2_ref[...].astype(bf16),
                 preferred_element_type=_f32) + be2_ref[...]
    wab = jnp.broadcast_to(wa_ref[...], (D, D)).astype(bf16)
    a_full = jnp.dot(ne.astype(bf16), wab,
                     preferred_element_type=_f32) + ba_ref[...]
    a_full = jnp.where(a_full >= 0, a_full, 0.2 * a_full)
    wf = jnp.exp(a_full)
    p_ref[...] = ne * wf
    ne_ref[...] = ne + ef
    wft = wf.T
    w_ref[...] = wft[0:1, :].reshape(1, 1, -1)


def _edge_mlp(g2, ef, we, we2, be2, wa, wat, ba):
    be = 2000
    grid = (E // be,)
    return pl.pallas_call(
        _k2_body,
        grid=grid,
        in_specs=[
            pl.BlockSpec((1, be, D), lambda i: (0, i, 0)),
            pl.BlockSpec((1, be, D), lambda i: (1, i, 0)),
            pl.BlockSpec((be, D), lambda i: (i, 0)),
            pl.BlockSpec((D, D), lambda i: (0, 0)),
            pl.BlockSpec((D, D), lambda i: (0, 0)),
            pl.BlockSpec((1, D), lambda i: (0, 0)),
            pl.BlockSpec((D, 1), lambda i: (0, 0)),
            pl.BlockSpec((1, D), lambda i: (0, 0)),
            pl.BlockSpec((1, 1), lambda i: (0, 0)),
        ],
        out_specs=[
            pl.BlockSpec((be, D), lambda i: (i, 0)),
            pl.BlockSpec((be, D), lambda i: (i, 0)),
            pl.BlockSpec((1, 1, be), lambda i: (i, 0, 0)),
        ],
        out_shape=[
            jax.ShapeDtypeStruct((E, D), _f32),
            jax.ShapeDtypeStruct((E, D), _f32),
            jax.ShapeDtypeStruct((E // be, 1, be), _f32),
        ],
    )(g2, g2, ef, we, we2, be2, wa, wat, ba)


def _k3_body(p_hbm, w_hbm, rcv_hbm, zs_hbm, zd_hbm, s_out, d_out,
             ridx0, prows0, wchunk0, ridx1, prows1, wchunk1,
             lsem0, lsem1, ssem0, ssem1, s_acc, d_acc):
    cid = lax.axis_index("c")
    sid = lax.axis_index("s")
    wid = sid * NC + cid
    rows_per_tile = NPAD // NS

    pltpu.sync_copy(zs_hbm.at[pl.ds(sid * rows_per_tile, rows_per_tile)],
                    s_acc.at[pl.ds(sid * rows_per_tile, rows_per_tile)])
    pltpu.sync_copy(zd_hbm.at[pl.ds(sid * rows_per_tile, rows_per_tile)],
                    d_acc.at[pl.ds(sid * rows_per_tile, rows_per_tile)])
    plsc.subcore_barrier()

    base = wid * EPW
    bufs = ((ridx0, prows0, wchunk0), (ridx1, prows1, wchunk1))
    lsems = (lsem0, lsem1)
    ssems = (ssem0, ssem1)

    def fire_load(j, b):
        ridx, prows, wchunk = bufs[b]
        off = base + j * W3
        pltpu.async_copy(rcv_hbm.at[pl.ds(off, W3)], ridx, lsems[b])
        pltpu.async_copy(p_hbm.at[pl.ds(off, W3)], prows, lsems[b])
        pltpu.async_copy(w_hbm.at[pl.ds(off, W3)], wchunk, lsems[b])

    def wait_load(j, b):
        ridx, prows, wchunk = bufs[b]
        off = base + j * W3
        pltpu.make_async_copy(rcv_hbm.at[pl.ds(off, W3)], ridx, lsems[b]).wait()
        pltpu.make_async_copy(p_hbm.at[pl.ds(off, W3)], prows, lsems[b]).wait()
        pltpu.make_async_copy(w_hbm.at[pl.ds(off, W3)], wchunk, lsems[b]).wait()

    def fire_scatter(b):
        ridx, prows, wchunk = bufs[b]
        pltpu.async_copy(prows, s_acc.at[ridx], ssems[b], add=True)
        pltpu.async_copy(wchunk, d_acc.at[ridx], ssems[b], add=True)

    def wait_scatter(b):
        ridx, prows, wchunk = bufs[b]
        pltpu.make_async_copy(prows, s_acc.at[ridx], ssems[b]).wait()
        pltpu.make_async_copy(wchunk, d_acc.at[ridx], ssems[b]).wait()

    fire_load(0, 0)

    def body(i, carry):
        j = i * 2
        @pl.when(j >= 1)
        def _():
            wait_scatter(1)
        @pl.when(j + 1 < NWIN3)
        def _():
            fire_load(j + 1, 1)
        wait_load(j, 0)
        fire_scatter(0)
        @pl.when(j + 1 < NWIN3)
        def _():
            wait_scatter(0)
            @pl.when(j + 2 < NWIN3)
            def _():
                fire_load(j + 2, 0)
            wait_load(j + 1, 1)
            fire_scatter(1)
        return carry

    lax.fori_loop(0, (NWIN3 + 1) // 2, body, 0)
    wait_scatter(0)
    if NWIN3 % 2 == 0:
        wait_scatter(1)
    plsc.subcore_barrier()

    pltpu.sync_copy(s_acc.at[pl.ds(sid * rows_per_tile, rows_per_tile)],
                    s_out.at[cid, pl.ds(sid * rows_per_tile, rows_per_tile)])
    pltpu.sync_copy(d_acc.at[pl.ds(sid * rows_per_tile, rows_per_tile)],
                    d_out.at[cid, pl.ds(sid * rows_per_tile, rows_per_tile)])


def _segment_sums(p, w, rcv):
    mesh = plsc.VectorSubcoreMesh(core_axis_name="c", subcore_axis_name="s",
                                  num_cores=NC, num_subcores=NS)
    zs = jnp.zeros((NPAD, D), _f32)
    zd = jnp.zeros((NPAD,), _f32)
    return pl.kernel(
        _k3_body,
        out_type=(
            jax.ShapeDtypeStruct((NC, NPAD, D), _f32),
            jax.ShapeDtypeStruct((NC, NPAD), _f32),
        ),
        mesh=mesh,
        scratch_types=[
            pltpu.VMEM((W3,), jnp.int32),
            pltpu.VMEM((W3, D), _f32),
            pltpu.VMEM((W3,), _f32),
            pltpu.VMEM((W3,), jnp.int32),
            pltpu.VMEM((W3, D), _f32),
            pltpu.VMEM((W3,), _f32),
            pltpu.SemaphoreType.DMA,
            pltpu.SemaphoreType.DMA,
            pltpu.SemaphoreType.DMA,
            pltpu.SemaphoreType.DMA,
            pltpu.VMEM_SHARED((NPAD, D), _f32),
            pltpu.VMEM_SHARED((NPAD,), _f32),
        ],
    )(p, w, rcv, zs, zd)


def _k4_body(nf_ref, s_ref, d_ref, wn1a_ref, wn1b_ref,
             bn1_ref, wn2_ref, bn2_ref, out_ref):
    x = nf_ref[...]
    d = d_ref[0] + d_ref[1] + 1e-16
    agg = (s_ref[0] + s_ref[1]) / d
    nh = (jnp.dot(x.astype(jnp.bfloat16), wn1a_ref[...].astype(jnp.bfloat16),
                  preferred_element_type=_f32)
          + jnp.dot(agg.astype(jnp.bfloat16),
                    wn1b_ref[...].astype(jnp.bfloat16),
                    preferred_element_type=_f32)
          + bn1_ref[...])
    nh = jnp.maximum(nh, 0.0)
    out_ref[...] = jnp.dot(nh.astype(jnp.bfloat16),
                           wn2_ref[...].astype(jnp.bfloat16),
                           preferred_element_type=_f32) + bn2_ref[...] + x


def _node_mlp(nf, s_part, d_part3, wn1a, wn1b, bn1, wn2, bn2):
    bn = 400
    grid = (N // bn,)
    return pl.pallas_call(
        _k4_body,
        grid=grid,
        in_specs=[
            pl.BlockSpec((bn, D), lambda i: (i, 0)),
            pl.BlockSpec((NC, bn, D), lambda i: (0, i, 0)),
            pl.BlockSpec((NC, bn, 1), lambda i: (0, i, 0)),
            pl.BlockSpec((D, D), lambda i: (0, 0)),
            pl.BlockSpec((D, D), lambda i: (0, 0)),
            pl.BlockSpec((1, D), lambda i: (0, 0)),
            pl.BlockSpec((D, D), lambda i: (0, 0)),
            pl.BlockSpec((1, D), lambda i: (0, 0)),
        ],
        out_specs=pl.BlockSpec((bn, D), lambda i: (i, 0)),
        out_shape=jax.ShapeDtypeStruct((N, D), _f32),
    )(nf, s_part, d_part3, wn1a, wn1b, bn1, wn2, bn2)


def kernel(node_features, edge_features, We1, be1, We2, be2,
           Wn1, bn1, Wn2, bn2, Wa, ba, senders, receivers):
    nf = node_features.reshape(N, D)
    ef = edge_features.reshape(E, D)
    snd = senders.astype(jnp.int32)
    rcv = receivers.astype(jnp.int32)

    ws = We1[:D]
    wr = We1[D:2 * D]
    we = We1[2 * D:]
    be1r = be1.reshape(1, D)
    be2r = be2.reshape(1, D)
    wat = Wa.reshape(1, D)
    bar = ba.reshape(1, 1)
    wn1a = Wn1[:D]
    wn1b = Wn1[D:]
    bn1r = bn1.reshape(1, D)
    bn2r = bn2.reshape(1, D)

    np2 = _node_proj(nf, ws, wr, be1r)
    sr = jnp.concatenate([snd, rcv], axis=0)
    g2 = _gather(np2, sr)
    ne, p, w = _edge_mlp(g2, ef, we, We2, be2r, Wa, wat, bar)
    s_part, d_part = _segment_sums(p, w.reshape(E), rcv)
    new_node = _node_mlp(nf, s_part, d_part.reshape(NC, NPAD, 1),
                         wn1a, wn1b, bn1r, Wn2, bn2r)

    return (new_node.reshape(1, N, D), ne.reshape(1, E, D))


# speedup vs baseline: 1.2776x; 1.0843x over previous
import functools

import jax
import jax.numpy as jnp
from jax import lax
from jax.experimental import pallas as pl
from jax.experimental.pallas import tpu as pltpu
from jax.experimental.pallas import tpu_sc as plsc

N = 10000
E = 320000
D = 128

NC = 2
NS = 16
NW = NC * NS
EPW = E // NW
W1 = 80
EPT1 = E // NS
NWIN1 = EPT1 // W1
W3 = 80
NWIN3 = EPW // W3
NPAD = 10240
HW = D // 2

_f32 = jnp.float32


def _k0_body(nf_ref, ws_ref, wr_ref, be1_ref, out_ref):
    x = nf_ref[...]
    out_ref[0] = jnp.dot(x, ws_ref[...], preferred_element_type=_f32)
    out_ref[1] = (jnp.dot(x, wr_ref[...], preferred_element_type=_f32)
                  + be1_ref[...])


def _node_proj(nf, ws, wr, be1):
    bn = 400
    grid = (N // bn,)
    return pl.pallas_call(
        _k0_body,
        grid=grid,
        in_specs=[
            pl.BlockSpec((bn, D), lambda i: (i, 0)),
            pl.BlockSpec((D, D), lambda i: (0, 0)),
            pl.BlockSpec((D, D), lambda i: (0, 0)),
            pl.BlockSpec((1, D), lambda i: (0, 0)),
        ],
        out_specs=pl.BlockSpec((NC, bn, D), lambda i: (0, i, 0)),
        out_shape=jax.ShapeDtypeStruct((NC, NPAD, D), _f32),
    )(nf, ws, wr, be1)


def _k1_body(np2_hbm, sr_hbm, g2_hbm,
             idx0, rows0, idx1, rows1, isem0, isem1, wsem0, wsem1, table):
    cid = lax.axis_index("c")
    sid = lax.axis_index("s")
    rpt = NPAD // NS

    pltpu.sync_copy(np2_hbm.at[cid, pl.ds(sid * rpt, rpt)],
                    table.at[pl.ds(sid * rpt, rpt)])
    plsc.subcore_barrier()

    base = sid * EPT1
    bufs = ((idx0, rows0), (idx1, rows1))
    isems = (isem0, isem1)
    wsems = (wsem0, wsem1)

    def fire_load_idx(j, b):
        pltpu.async_copy(sr_hbm.at[pl.ds(cid * E + base + j * W1, W1)],
                         bufs[b][0], isems[b])

    def gather(b):
        idx, rows = bufs[b]
        pltpu.make_async_copy(sr_hbm.at[pl.ds(cid * E + base, W1)],
                              idx, isems[b]).wait()
        pltpu.sync_copy(table.at[idx], rows)

    def fire_wb(j, b):
        pltpu.async_copy(bufs[b][1],
                         g2_hbm.at[cid, pl.ds(base + j * W1, W1)], wsems[b])

    def wait_wb(j, b):
        pltpu.make_async_copy(bufs[b][1],
                              g2_hbm.at[cid, pl.ds(base + j * W1, W1)],
                              wsems[b]).wait()

    fire_load_idx(0, 0)

    def body(i, carry):
        j = i * 2
        fire_load_idx(j + 1, 1)
        gather(0)
        fire_wb(j, 0)
        @pl.when(j + 2 < NWIN1)
        def _():
            fire_load_idx(j + 2, 0)
        gather(1)
        wait_wb(j, 0)
        fire_wb(j + 1, 1)
        wait_wb(j + 1, 1)
        return carry

    lax.fori_loop(0, NWIN1 // 2, body, 0)


def _gather(np2, sr):
    mesh = plsc.VectorSubcoreMesh(core_axis_name="c", subcore_axis_name="s",
                                  num_cores=NC, num_subcores=NS)
    return pl.kernel(
        _k1_body,
        out_type=jax.ShapeDtypeStruct((NC, E, D), _f32),
        mesh=mesh,
        scratch_types=[
            pltpu.VMEM((W1,), jnp.int32),
            pltpu.VMEM((W1, D), _f32),
            pltpu.VMEM((W1,), jnp.int32),
            pltpu.VMEM((W1, D), _f32),
            pltpu.SemaphoreType.DMA,
            pltpu.SemaphoreType.DMA,
            pltpu.SemaphoreType.DMA,
            pltpu.SemaphoreType.DMA,
            pltpu.VMEM_SHARED((NPAD, D), _f32),
        ],
    )(np2, sr)


def _k2_body(gs_ref, gr_ref, ef_ref, we_ref, we2_ref, be2_ref, wa_ref,
             wat_ref, ba_ref, ne_ref, p_ref, w_ref):
    bf16 = jnp.bfloat16
    ef = ef_ref[...]
    h = gs_ref[0] + gr_ref[0] + jnp.dot(
        ef.astype(bf16), we_ref[...].astype(bf16), preferred_element_type=_f32)
    h = jnp.maximum(h, 0.0)
    ne = jnp.dot(h.astype(bf16), we2_ref[...].astype(bf16),
                 preferred_element_type=_f32) + be2_ref[...]
    wab = jnp.broadcast_to(wa_ref[...], (D, D)).astype(bf16)
    a_full = jnp.dot(ne.astype(bf16), wab,
                     preferred_element_type=_f32) + ba_ref[...]
    a_full = jnp.where(a_full >= 0, a_full, 0.2 * a_full)
    wf = jnp.exp(a_full)
    p_ref[...] = ne * wf
    ne_ref[...] = ne + ef
    wft = wf.T
    w_ref[...] = wft[0:1, :].reshape(1, 1, -1)


def _edge_mlp(g2, ef, we, we2, be2, wa, wat, ba):
    be = 4000
    grid = (E // be,)
    return pl.pallas_call(
        _k2_body,
        grid=grid,
        in_specs=[
            pl.BlockSpec((1, be, D), lambda i: (0, i, 0)),
            pl.BlockSpec((1, be, D), lambda i: (1, i, 0)),
            pl.BlockSpec((be, D), lambda i: (i, 0)),
            pl.BlockSpec((D, D), lambda i: (0, 0)),
            pl.BlockSpec((D, D), lambda i: (0, 0)),
            pl.BlockSpec((1, D), lambda i: (0, 0)),
            pl.BlockSpec((D, 1), lambda i: (0, 0)),
            pl.BlockSpec((1, D), lambda i: (0, 0)),
            pl.BlockSpec((1, 1), lambda i: (0, 0)),
        ],
        out_specs=[
            pl.BlockSpec((be, D), lambda i: (i, 0)),
            pl.BlockSpec((be, D), lambda i: (i, 0)),
            pl.BlockSpec((1, 1, be), lambda i: (i, 0, 0)),
        ],
        out_shape=[
            jax.ShapeDtypeStruct((E, D), _f32),
            jax.ShapeDtypeStruct((E, D), _f32),
            jax.ShapeDtypeStruct((E // be, 1, be), _f32),
        ],
    )(g2, g2, ef, we, we2, be2, wa, wat, ba)


def _k3_body(p_hbm, w_hbm, rcv_hbm, zs_hbm, zd_hbm, s_out, d_out,
             ridx0, prows0, wchunk0, ridx1, prows1, wchunk1,
             lsem0, lsem1, ssem0, ssem1, s_acc, d_acc):
    cid = lax.axis_index("c")
    sid = lax.axis_index("s")
    wid = sid * NC + cid
    rows_per_tile = NPAD // NS

    pltpu.sync_copy(zs_hbm.at[pl.ds(sid * rows_per_tile, rows_per_tile)],
                    s_acc.at[pl.ds(sid * rows_per_tile, rows_per_tile)])
    pltpu.sync_copy(zd_hbm.at[pl.ds(sid * rows_per_tile, rows_per_tile)],
                    d_acc.at[pl.ds(sid * rows_per_tile, rows_per_tile)])
    plsc.subcore_barrier()

    base = wid * EPW
    bufs = ((ridx0, prows0, wchunk0), (ridx1, prows1, wchunk1))
    lsems = (lsem0, lsem1)
    ssems = (ssem0, ssem1)

    def fire_load(j, b):
        ridx, prows, wchunk = bufs[b]
        off = base + j * W3
        pltpu.async_copy(rcv_hbm.at[pl.ds(off, W3)], ridx, lsems[b])
        pltpu.async_copy(p_hbm.at[pl.ds(off, W3)], prows, lsems[b])
        pltpu.async_copy(w_hbm.at[pl.ds(off, W3)], wchunk, lsems[b])

    def wait_load(j, b):
        ridx, prows, wchunk = bufs[b]
        off = base + j * W3
        pltpu.make_async_copy(rcv_hbm.at[pl.ds(off, W3)], ridx, lsems[b]).wait()
        pltpu.make_async_copy(p_hbm.at[pl.ds(off, W3)], prows, lsems[b]).wait()
        pltpu.make_async_copy(w_hbm.at[pl.ds(off, W3)], wchunk, lsems[b]).wait()

    def fire_scatter(b):
        ridx, prows, wchunk = bufs[b]
        pltpu.async_copy(prows, s_acc.at[ridx], ssems[b], add=True)
        pltpu.async_copy(wchunk, d_acc.at[ridx], ssems[b], add=True)

    def wait_scatter(b):
        ridx, prows, wchunk = bufs[b]
        pltpu.make_async_copy(prows, s_acc.at[ridx], ssems[b]).wait()
        pltpu.make_async_copy(wchunk, d_acc.at[ridx], ssems[b]).wait()

    fire_load(0, 0)

    def body(i, carry):
        j = i * 2
        @pl.when(j >= 1)
        def _():
            wait_scatter(1)
        @pl.when(j + 1 < NWIN3)
        def _():
            fire_load(j + 1, 1)
        wait_load(j, 0)
        fire_scatter(0)
        @pl.when(j + 1 < NWIN3)
        def _():
            wait_scatter(0)
            @pl.when(j + 2 < NWIN3)
            def _():
                fire_load(j + 2, 0)
            wait_load(j + 1, 1)
            fire_scatter(1)
        return carry

    lax.fori_loop(0, (NWIN3 + 1) // 2, body, 0)
    wait_scatter(0)
    if NWIN3 % 2 == 0:
        wait_scatter(1)
    plsc.subcore_barrier()

    pltpu.sync_copy(s_acc.at[pl.ds(sid * rows_per_tile, rows_per_tile)],
                    s_out.at[cid, pl.ds(sid * rows_per_tile, rows_per_tile)])
    pltpu.sync_copy(d_acc.at[pl.ds(sid * rows_per_tile, rows_per_tile)],
                    d_out.at[cid, pl.ds(sid * rows_per_tile, rows_per_tile)])


def _segment_sums(p, w, rcv):
    mesh = plsc.VectorSubcoreMesh(core_axis_name="c", subcore_axis_name="s",
                                  num_cores=NC, num_subcores=NS)
    zs = jnp.zeros((NPAD, D), _f32)
    zd = jnp.zeros((NPAD,), _f32)
    return pl.kernel(
        _k3_body,
        out_type=(
            jax.ShapeDtypeStruct((NC, NPAD, D), _f32),
            jax.ShapeDtypeStruct((NC, NPAD), _f32),
        ),
        mesh=mesh,
        scratch_types=[
            pltpu.VMEM((W3,), jnp.int32),
            pltpu.VMEM((W3, D), _f32),
            pltpu.VMEM((W3,), _f32),
            pltpu.VMEM((W3,), jnp.int32),
            pltpu.VMEM((W3, D), _f32),
            pltpu.VMEM((W3,), _f32),
            pltpu.SemaphoreType.DMA,
            pltpu.SemaphoreType.DMA,
            pltpu.SemaphoreType.DMA,
            pltpu.SemaphoreType.DMA,
            pltpu.VMEM_SHARED((NPAD, D), _f32),
            pltpu.VMEM_SHARED((NPAD,), _f32),
        ],
    )(p, w, rcv, zs, zd)


def _k4_body(nf_ref, s_ref, d_ref, wn1a_ref, wn1b_ref,
             bn1_ref, wn2_ref, bn2_ref, out_ref):
    x = nf_ref[...]
    d = d_ref[0] + d_ref[1] + 1e-16
    agg = (s_ref[0] + s_ref[1]) / d
    nh = (jnp.dot(x.astype(jnp.bfloat16), wn1a_ref[...].astype(jnp.bfloat16),
                  preferred_element_type=_f32)
          + jnp.dot(agg.astype(jnp.bfloat16),
                    wn1b_ref[...].astype(jnp.bfloat16),
                    preferred_element_type=_f32)
          + bn1_ref[...])
    nh = jnp.maximum(nh, 0.0)
    out_ref[...] = jnp.dot(nh.astype(jnp.bfloat16),
                           wn2_ref[...].astype(jnp.bfloat16),
                           preferred_element_type=_f32) + bn2_ref[...] + x


def _node_mlp(nf, s_part, d_part3, wn1a, wn1b, bn1, wn2, bn2):
    bn = 400
    grid = (N // bn,)
    return pl.pallas_call(
        _k4_body,
        grid=grid,
        in_specs=[
            pl.BlockSpec((bn, D), lambda i: (i, 0)),
            pl.BlockSpec((NC, bn, D), lambda i: (0, i, 0)),
            pl.BlockSpec((NC, bn, 1), lambda i: (0, i, 0)),
            pl.BlockSpec((D, D), lambda i: (0, 0)),
            pl.BlockSpec((D, D), lambda i: (0, 0)),
            pl.BlockSpec((1, D), lambda i: (0, 0)),
            pl.BlockSpec((D, D), lambda i: (0, 0)),
            pl.BlockSpec((1, D), lambda i: (0, 0)),
        ],
        out_specs=pl.BlockSpec((bn, D), lambda i: (i, 0)),
        out_shape=jax.ShapeDtypeStruct((N, D), _f32),
    )(nf, s_part, d_part3, wn1a, wn1b, bn1, wn2, bn2)


def kernel(node_features, edge_features, We1, be1, We2, be2,
           Wn1, bn1, Wn2, bn2, Wa, ba, senders, receivers):
    nf = node_features.reshape(N, D)
    ef = edge_features.reshape(E, D)
    snd = senders.astype(jnp.int32)
    rcv = receivers.astype(jnp.int32)

    ws = We1[:D]
    wr = We1[D:2 * D]
    we = We1[2 * D:]
    be1r = be1.reshape(1, D)
    be2r = be2.reshape(1, D)
    wat = Wa.reshape(1, D)
    bar = ba.reshape(1, 1)
    wn1a = Wn1[:D]
    wn1b = Wn1[D:]
    bn1r = bn1.reshape(1, D)
    bn2r = bn2.reshape(1, D)

    np2 = _node_proj(nf, ws, wr, be1r)
    sr = jnp.concatenate([snd, rcv], axis=0)
    g2 = _gather(np2, sr)
    ne, p, w = _edge_mlp(g2, ef, we, We2, be2r, Wa, wat, bar)
    s_part, d_part = _segment_sums(p, w.reshape(E), rcv)
    new_node = _node_mlp(nf, s_part, d_part.reshape(NC, NPAD, 1),
                         wn1a, wn1b, bn1r, Wn2, bn2r)

    return (new_node.reshape(1, N, D), ne.reshape(1, E, D))
